# baseline (speedup 1.0000x reference)
import functools

import jax
import jax.numpy as jnp
from jax import lax
from jax.experimental import pallas as pl
from jax.experimental.pallas import tpu as pltpu
from jax.experimental.pallas import tpu_sc as plsc

NF = 19
NI = 128
NO = 256
VOCAB = 1000

NC = 2
NS = 16
NW = NC * NS
LANES = 16
NU = NO // 2

CH = 4
FA = 10
FB = NF - FA
ROWS_A = CH * FA
ROWS_B = CH * FB
ROWSP = 40
NBUF = 2


def _proj_body(emb_ref, w_ref, be_ref, bo_ref, p_ref):
    f = pl.program_id(0)
    e16 = emb_ref[0].astype(jnp.bfloat16)
    w16 = w_ref[0].astype(jnp.bfloat16)
    acc = jnp.dot(e16, w16, preferred_element_type=jnp.float32)
    is0 = (f == 0).astype(jnp.float32)
    acc_e = acc[:, :NU] + is0 * be_ref[0][None, :]
    acc_o = acc[:, NU:] + is0 * bo_ref[0][None, :]
    be = lax.bitcast_convert_type(acc_e, jnp.uint32) + jnp.uint32(0x8000)
    bo = lax.bitcast_convert_type(acc_o, jnp.uint32) + jnp.uint32(0x8000)
    p_ref[0] = (be >> 16) | (bo & jnp.uint32(0xFFFF0000))


def _project_tables(emb_tables, fc_W, fc_b):
    w3 = fc_W.reshape(NF, NI, NO)
    b_e = fc_b[:NU].reshape(1, NU)
    b_o = fc_b[NU:].reshape(1, NU)
    p = pl.pallas_call(
        _proj_body,
        grid=(NF,),
        in_specs=[
            pl.BlockSpec((1, VOCAB, NI), lambda f: (f, 0, 0)),
            pl.BlockSpec((1, NI, NO), lambda f: (f, 0, 0)),
            pl.BlockSpec((1, NU), lambda f: (0, 0)),
            pl.BlockSpec((1, NU), lambda f: (0, 0)),
        ],
        out_specs=pl.BlockSpec((1, VOCAB, NU), lambda f: (f, 0, 0)),
        out_shape=jax.ShapeDtypeStruct((NF, VOCAB, NU), jnp.uint32),
    )(emb_tables, w3, b_e, b_o)
    return p.reshape(NF * VOCAB, NU)


def _make_bag_kernel(bs):
    spw = bs // NW
    nchunk = spw // CH
    mesh = plsc.VectorSubcoreMesh(core_axis_name="c", subcore_axis_name="s")

    @functools.partial(
        pl.kernel,
        out_type=jax.ShapeDtypeStruct((bs, NO), jnp.float32),
        mesh=mesh,
        scratch_types=[
            pltpu.VMEM((nchunk, ROWSP), jnp.int32),
            pltpu.VMEM((ROWSP, NU), jnp.uint32),
            pltpu.VMEM((ROWSP, NU), jnp.uint32),
            pltpu.VMEM((spw, NO), jnp.float32),
            pltpu.VMEM_SHARED((FA * VOCAB, NU), jnp.uint32),
            pltpu.SemaphoreType.DMA,
            pltpu.SemaphoreType.DMA,
        ],
    )
    def bag(idxa_hbm, idxb_hbm, p_hbm, out_hbm, idx_v,
            rows0, rows1, out_v, psh, sem0, sem1):
        cid = lax.axis_index("c")
        sid = lax.axis_index("s")
        w = sid * NC + cid

        bufs = (rows0, rows1)
        sems = (sem0, sem1)
        himask = jnp.full((LANES,), 0xFFFF0000, jnp.uint32)

        def stage(nrows, hbm_off):
            per = (nrows // (NS * 8)) * 8
            tail = nrows - NS * per
            npiece = per // ROWSP
            rem = per - npiece * ROWSP
            base_h = hbm_off + sid * per
            base_s = sid * per

            def fetch(g, buf, sem):
                pltpu.async_copy(
                    p_hbm.at[pl.ds(base_h + g * ROWSP, ROWSP)], buf, sem)

            def fwait(g, buf, sem):
                pltpu.make_async_copy(
                    p_hbm.at[pl.ds(base_h + g * ROWSP, ROWSP)], buf,
                    sem).wait()

            def push(g, buf):
                pltpu.sync_copy(buf, psh.at[pl.ds(base_s + g * ROWSP, ROWSP)])

            fetch(0, rows0, sem0)

            def body(h, carry):
                g = 2 * h

                @pl.when(g + 1 < npiece)
                def _():
                    fetch(g + 1, rows1, sem1)

                fwait(g, rows0, sem0)
                push(g, rows0)

                @pl.when(g + 2 < npiece)
                def _():
                    fetch(g + 2, rows0, sem0)

                @pl.when(g + 1 < npiece)
                def _():
                    fwait(g + 1, rows1, sem1)
                    push(g + 1, rows1)

                return carry

            lax.fori_loop(0, (npiece + 1) // 2, body, 0)

            def hop(hbm_off_g, psh_off_g, n):
                src = p_hbm.at[pl.ds(hbm_off_g, n)]
                mid = rows0 if n == ROWSP else rows0.at[pl.ds(0, n)]
                pltpu.sync_copy(src, mid)
                pltpu.sync_copy(mid, psh.at[pl.ds(psh_off_g, n)])

            if rem:
                hop(base_h + npiece * ROWSP, base_s + npiece * ROWSP, rem)

            @pl.when(sid == NS - 1)
            def _():
                hop(hbm_off + NS * per, NS * per, tail)

        def make_compute(fpp, accumulate):
            def compute_chunk(j, rows):
                for cc in range(NU // LANES):
                    col = pl.ds(cc * LANES, LANES)
                    colh = pl.ds(NU + cc * LANES, LANES)
                    for s in range(CH):
                        x = rows[s * fpp, col]
                        a0 = lax.bitcast_convert_type(x << 16, jnp.float32)
                        a1 = lax.bitcast_convert_type(x & himask, jnp.float32)
                        for f in range(1, fpp):
                            x = rows[s * fpp + f, col]
                            a0 = a0 + lax.bitcast_convert_type(
                                x << 16, jnp.float32)
                            a1 = a1 + lax.bitcast_convert_type(
                                x & himask, jnp.float32)
                        r = j * CH + s
                        if accumulate:
                            a0 = a0 + out_v[r, col]
                            a1 = a1 + out_v[r, colh]
                        out_v[r, col] = a0
                        out_v[r, colh] = a1
            return compute_chunk

        def run_pass(idx_v, compute_chunk):
            def gather(j, b):
                pltpu.async_copy(psh.at[idx_v.at[j]], bufs[b], sems[b])

            def wait(j, b):
                pltpu.make_async_copy(
                    psh.at[idx_v.at[j]], bufs[b], sems[b]).wait()

            for b in range(NBUF):
                gather(b, b)

            def loop_body(g, carry):
                base = g * NBUF
                for b in range(NBUF):
                    j = base + b
                    wait(j, b)
                    compute_chunk(j, bufs[b])

                    @pl.when(j + NBUF < nchunk)
                    def _():
                        gather(j + NBUF, b)
                return carry

            lax.fori_loop(0, nchunk // NBUF, loop_body, 0)

        stage(FA * VOCAB, 0)
        pltpu.sync_copy(idxa_hbm.at[w], idx_v)
        plsc.subcore_barrier()
        run_pass(idx_v, make_compute(FA, accumulate=False))
        plsc.subcore_barrier()

        stage(FB * VOCAB, FA * VOCAB)
        pltpu.sync_copy(idxb_hbm.at[w], idx_v)
        plsc.subcore_barrier()
        run_pass(idx_v, make_compute(FB, accumulate=True))

        pltpu.sync_copy(out_v, out_hbm.at[pl.ds(w * spw, spw)])

    return bag


def kernel(feature, edge_index, emb_tables, fc_W, fc_b):
    del edge_index
    bs = feature.shape[0]
    spw = bs // NW
    nchunk = spw // CH

    fi = feature.astype(jnp.int32)
    offs = jnp.arange(NF, dtype=jnp.int32) * VOCAB
    idxa = (fi[:, :FA] + offs[None, :FA]).reshape(NW, nchunk, ROWS_A)
    idxb = (fi[:, FA:] + (offs[None, FA:] - FA * VOCAB)).reshape(
        NW, nchunk, ROWS_B)
    idxb = jnp.pad(idxb, ((0, 0), (0, 0), (0, ROWSP - ROWS_B)))

    p = _project_tables(emb_tables, fc_W, fc_b)
    return _make_bag_kernel(bs)(idxa, idxb, p)


# speedup vs baseline: 1.4066x; 1.4066x over previous
import functools

import jax
import jax.numpy as jnp
from jax import lax
from jax.experimental import pallas as pl
from jax.experimental.pallas import tpu as pltpu
from jax.experimental.pallas import tpu_sc as plsc

NF = 19
NI = 128
NO = 256
VOCAB = 1000

NC = 2
NS = 16
NW = NC * NS
LANES = 16
NU = NO // 2

CH = 4
FA = 10
FB = NF - FA
ROWS_A = CH * FA
ROWS_B = CH * FB
ROWSP = 40
NBUF = 2


def _proj_body(emb_ref, w_ref, be_ref, bo_ref, p_ref):
    f = pl.program_id(0)
    e16 = emb_ref[0].astype(jnp.bfloat16)
    w16 = w_ref[0].astype(jnp.bfloat16)
    acc = jnp.dot(e16, w16, preferred_element_type=jnp.float32)
    is0 = (f == 0).astype(jnp.float32)
    acc_e = acc[:, :NU] + is0 * be_ref[0][None, :]
    acc_o = acc[:, NU:] + is0 * bo_ref[0][None, :]
    be = lax.bitcast_convert_type(acc_e, jnp.uint32) + jnp.uint32(0x8000)
    bo = lax.bitcast_convert_type(acc_o, jnp.uint32) + jnp.uint32(0x8000)
    p_ref[0] = (be >> 16) | (bo & jnp.uint32(0xFFFF0000))


def _project_tables(emb_tables, fc_W, fc_b):
    w3 = fc_W.reshape(NF, NI, NO)
    b_e = fc_b[:NU].reshape(1, NU)
    b_o = fc_b[NU:].reshape(1, NU)
    p = pl.pallas_call(
        _proj_body,
        grid=(NF,),
        in_specs=[
            pl.BlockSpec((1, VOCAB, NI), lambda f: (f, 0, 0)),
            pl.BlockSpec((1, NI, NO), lambda f: (f, 0, 0)),
            pl.BlockSpec((1, NU), lambda f: (0, 0)),
            pl.BlockSpec((1, NU), lambda f: (0, 0)),
        ],
        out_specs=pl.BlockSpec((1, VOCAB, NU), lambda f: (f, 0, 0)),
        out_shape=jax.ShapeDtypeStruct((NF, VOCAB, NU), jnp.uint32),
    )(emb_tables, w3, b_e, b_o)
    return p.reshape(NF * VOCAB, NU)


def _make_bag_kernel(bs):
    spw = bs // NW
    nchunk = spw // CH
    mesh = plsc.VectorSubcoreMesh(core_axis_name="c", subcore_axis_name="s")

    @functools.partial(
        pl.kernel,
        out_type=jax.ShapeDtypeStruct((bs, NO), jnp.float32),
        mesh=mesh,
        scratch_types=[
            pltpu.VMEM((nchunk, ROWSP), jnp.int32),
            pltpu.VMEM((ROWSP, NU), jnp.uint32),
            pltpu.VMEM((ROWSP, NU), jnp.uint32),
            pltpu.VMEM((spw, NO), jnp.float32),
            pltpu.VMEM_SHARED((FA * VOCAB, NU), jnp.uint32),
            pltpu.SemaphoreType.DMA,
            pltpu.SemaphoreType.DMA,
        ],
    )
    def bag(idxa_hbm, idxb_hbm, p_hbm, out_hbm, idx_v,
            rows0, rows1, out_v, psh, sem0, sem1):
        cid = lax.axis_index("c")
        sid = lax.axis_index("s")
        w = sid * NC + cid

        bufs = (rows0, rows1)
        sems = (sem0, sem1)
        himask = jnp.full((LANES,), 0xFFFF0000, jnp.uint32)

        def stage(nrows, hbm_off):
            per = (nrows // (NS * 8)) * 8
            tail = nrows - NS * per
            npiece = per // ROWSP
            rem = per - npiece * ROWSP
            base_h = hbm_off + sid * per
            base_s = sid * per

            def fetch(g, buf, sem):
                pltpu.async_copy(
                    p_hbm.at[pl.ds(base_h + g * ROWSP, ROWSP)], buf, sem)

            def fwait(g, buf, sem):
                pltpu.make_async_copy(
                    p_hbm.at[pl.ds(base_h + g * ROWSP, ROWSP)], buf,
                    sem).wait()

            def push(g, buf):
                pltpu.sync_copy(buf, psh.at[pl.ds(base_s + g * ROWSP, ROWSP)])

            fetch(0, rows0, sem0)

            def body(h, carry):
                g = 2 * h

                @pl.when(g + 1 < npiece)
                def _():
                    fetch(g + 1, rows1, sem1)

                fwait(g, rows0, sem0)
                push(g, rows0)

                @pl.when(g + 2 < npiece)
                def _():
                    fetch(g + 2, rows0, sem0)

                @pl.when(g + 1 < npiece)
                def _():
                    fwait(g + 1, rows1, sem1)
                    push(g + 1, rows1)

                return carry

            lax.fori_loop(0, (npiece + 1) // 2, body, 0)

            def hop(hbm_off_g, psh_off_g, n):
                src = p_hbm.at[pl.ds(hbm_off_g, n)]
                mid = rows0 if n == ROWSP else rows0.at[pl.ds(0, n)]
                pltpu.sync_copy(src, mid)
                pltpu.sync_copy(mid, psh.at[pl.ds(psh_off_g, n)])

            if rem:
                hop(base_h + npiece * ROWSP, base_s + npiece * ROWSP, rem)

            @pl.when(sid == NS - 1)
            def _():
                hop(hbm_off + NS * per, NS * per, tail)

        def make_compute(fpp, accumulate):
            def compute_chunk(j, rows):
                def cc_body(cc, carry):
                    col = pl.ds(cc * LANES, LANES)
                    colh = pl.ds(NU + cc * LANES, LANES)
                    for s in range(CH):
                        x = rows[s * fpp, col]
                        a0 = lax.bitcast_convert_type(x << 16, jnp.float32)
                        a1 = lax.bitcast_convert_type(x & himask, jnp.float32)
                        for f in range(1, fpp):
                            x = rows[s * fpp + f, col]
                            a0 = a0 + lax.bitcast_convert_type(
                                x << 16, jnp.float32)
                            a1 = a1 + lax.bitcast_convert_type(
                                x & himask, jnp.float32)
                        r = j * CH + s
                        if accumulate:
                            a0 = a0 + out_v[r, col]
                            a1 = a1 + out_v[r, colh]
                        out_v[r, col] = a0
                        out_v[r, colh] = a1
                    return carry

                lax.fori_loop(0, NU // LANES, cc_body, 0)
            return compute_chunk

        def run_pass(idx_v, compute_chunk):
            def gather(j, b):
                pltpu.async_copy(psh.at[idx_v.at[j]], bufs[b], sems[b])

            def wait(j, b):
                pltpu.make_async_copy(
                    psh.at[idx_v.at[j]], bufs[b], sems[b]).wait()

            for b in range(NBUF):
                gather(b, b)

            def loop_body(g, carry):
                base = g * NBUF
                for b in range(NBUF):
                    j = base + b
                    wait(j, b)
                    compute_chunk(j, bufs[b])

                    @pl.when(j + NBUF < nchunk)
                    def _():
                        gather(j + NBUF, b)
                return carry

            lax.fori_loop(0, nchunk // NBUF, loop_body, 0)

        stage(FA * VOCAB, 0)
        pltpu.sync_copy(idxa_hbm.at[w], idx_v)
        plsc.subcore_barrier()
        run_pass(idx_v, make_compute(FA, accumulate=False))
        plsc.subcore_barrier()

        stage(FB * VOCAB, FA * VOCAB)
        pltpu.sync_copy(idxb_hbm.at[w], idx_v)
        plsc.subcore_barrier()
        run_pass(idx_v, make_compute(FB, accumulate=True))

        pltpu.sync_copy(out_v, out_hbm.at[pl.ds(w * spw, spw)])

    return bag


def kernel(feature, edge_index, emb_tables, fc_W, fc_b):
    del edge_index
    bs = feature.shape[0]
    spw = bs // NW
    nchunk = spw // CH

    fi = feature.astype(jnp.int32)
    offs = jnp.arange(NF, dtype=jnp.int32) * VOCAB
    idxa = (fi[:, :FA] + offs[None, :FA]).reshape(NW, nchunk, ROWS_A)
    idxb = (fi[:, FA:] + (offs[None, FA:] - FA * VOCAB)).reshape(
        NW, nchunk, ROWS_B)
    idxb = jnp.pad(idxb, ((0, 0), (0, 0), (0, ROWSP - ROWS_B)))

    p = _project_tables(emb_tables, fc_W, fc_b)
    return _make_bag_kernel(bs)(idxa, idxb, p)
